# baseline (speedup 1.0000x reference)
import jax
import jax.numpy as jnp
from jax import lax
from jax.experimental import pallas as pl
from jax.experimental.pallas import tpu as pltpu
from jax.experimental.pallas import tpu_sc as plsc

B, C, H, W, NUM = 4, 256, 64, 64, 8
HW = H * W
P = B * HW
NCAND = 3 * NUM
K = NCAND // 3
NWORKERS = 32
PIX_PER_W = P // NWORKERS
CHUNK = 8
NCHUNKS = PIX_PER_W // CHUNK
ROWS = CHUNK * NCAND
HROWS = ROWS // 2
NEG = -1e30
FLT_MIN = 1.1754944e-38
LOG_FLT_MIN = -87.336545


def _sc_evaluate_kernel(rt_hbm, inds_hbm, left_hbm, corr_out, inds_out,
                        idx3, rows2, left2,
                        outc_v, outi_v, sem0, sem1):
    wid = lax.axis_index("s") * 2 + lax.axis_index("c")
    bbase = (wid // 8) * HW
    iota16 = lax.iota(jnp.int32, 16)
    sems = (sem0, sem1)

    def issue(ch, b):
        pix_base = wid * PIX_PER_W + ch * CHUNK
        pltpu.sync_copy(inds_hbm.at[wid * NCHUNKS + ch], idx3.at[b])
        pltpu.async_copy(left_hbm.at[pl.ds(pix_base, CHUNK)], left2.at[b],
                         sems[b])
        pltpu.async_copy(rt_hbm.at[idx3.at[b].at[0]],
                         rows2.at[b].at[pl.ds(0, HROWS)], sems[b])
        pltpu.async_copy(rt_hbm.at[idx3.at[b].at[1]],
                         rows2.at[b].at[pl.ds(HROWS, HROWS)], sems[b])

    def drain(b):
        pltpu.make_async_copy(left_hbm.at[pl.ds(0, CHUNK)], left2.at[b],
                              sems[b]).wait()
        pltpu.make_async_copy(rt_hbm.at[idx3.at[b].at[0]],
                              rows2.at[b].at[pl.ds(0, HROWS)], sems[b]).wait()
        pltpu.make_async_copy(rt_hbm.at[idx3.at[b].at[1]],
                              rows2.at[b].at[pl.ds(HROWS, HROWS)],
                              sems[b]).wait()

    def compute(ch, b):
        pix_base = wid * PIX_PER_W + ch * CHUNK

        def pix_body(p, carry2):
            lvecs = [left2[b, p, pl.ds(v * 16, 16)] for v in range(16)]
            c0 = jnp.zeros((16,), jnp.float32)
            c1 = jnp.full((16,), NEG, jnp.float32)
            for cand in range(NCAND):
                row = p * NCAND + cand
                acc = lvecs[0] * rows2[b, row, pl.ds(0, 16)]
                for v in range(1, 16):
                    acc = acc + lvecs[v] * rows2[b, row, pl.ds(v * 16, 16)]
                s = jnp.sum(acc) * 100.0
                if cand < 16:
                    c0 = jnp.where(iota16 == cand, s, c0)
                else:
                    c1 = jnp.where(iota16 == cand - 16, s, c1)
            m = jnp.maximum(jnp.max(c0), jnp.max(c1))
            d0 = c0 - m
            d1 = c1 - m
            e0 = jnp.where(d0 < LOG_FLT_MIN, 0.0, jnp.exp(d0))
            e1 = jnp.where(d1 < LOG_FLT_MIN, 0.0, jnp.exp(d1))
            svec = jnp.zeros((16,), jnp.float32) + (jnp.sum(e0) + jnp.sum(e1))
            p0 = e0 / svec
            p1 = e1 / svec
            p0 = jnp.where(p0 < FLT_MIN, 0.0, p0)
            p1 = jnp.where(p1 < FLT_MIN, 0.0, p1)

            topv = jnp.zeros((16,), jnp.float32)
            topi = jnp.zeros((16,), jnp.int32)
            for k in range(K):
                mx = jnp.max(jnp.maximum(p0, p1))
                g0 = jnp.where(p0 == mx, iota16, 64)
                g1 = jnp.where(p1 == mx, iota16 + 16, 64)
                idx = jnp.min(jnp.minimum(g0, g1))
                topv = jnp.where(iota16 == k, mx, topv)
                topi = jnp.where(iota16 == k, idx, topi)
                p0 = jnp.where(iota16 == idx, -1.0, p0)
                p1 = jnp.where(iota16 == idx - 16, -1.0, p1)

            f = p * NCAND + topi
            ga = plsc.load_gather(idx3.at[b].at[0],
                                  [jnp.minimum(f, HROWS - 1)])
            gb = plsc.load_gather(idx3.at[b].at[1],
                                  [jnp.maximum(f - HROWS, 0)])
            g = jnp.where(f < HROWS, ga, gb)
            outi_v[p, :] = (g - bbase).astype(jnp.float32)
            outc_v[p, :] = topv
            return carry2

        lax.fori_loop(0, CHUNK, pix_body, 0, unroll=2)
        pltpu.sync_copy(outc_v, corr_out.at[pl.ds(pix_base, CHUNK)])
        pltpu.sync_copy(outi_v, inds_out.at[pl.ds(pix_base, CHUNK)])

    issue(0, 0)

    def pair_body(gidx, carry):
        che = 2 * gidx
        issue(che + 1, 1)
        drain(0)
        compute(che, 0)
        issue(jnp.minimum(che + 2, NCHUNKS - 2), 0)
        drain(1)
        compute(che + 1, 1)
        return carry

    lax.fori_loop(0, NCHUNKS // 2, pair_body, 0)
    drain(0)


@jax.jit
def _sc_evaluate(rt, inds_flat, leftr):
    mesh = plsc.VectorSubcoreMesh(core_axis_name="c", subcore_axis_name="s")
    fn = pl.kernel(
        _sc_evaluate_kernel,
        mesh=mesh,
        compiler_params=pltpu.CompilerParams(needs_layout_passes=False),
        out_type=[
            jax.ShapeDtypeStruct((P, 16), jnp.float32),
            jax.ShapeDtypeStruct((P, 16), jnp.float32),
        ],
        scratch_types=[
            pltpu.VMEM((2, 2, HROWS), jnp.int32),
            pltpu.VMEM((2, ROWS, C), jnp.float32),
            pltpu.VMEM((2, CHUNK, C), jnp.float32),
            pltpu.VMEM((CHUNK, 16), jnp.float32),
            pltpu.VMEM((CHUNK, 16), jnp.float32),
            pltpu.SemaphoreType.DMA,
            pltpu.SemaphoreType.DMA,
        ],
    )
    return fn(rt, inds_flat, leftr)


def _propagation(ox, oy, ptype):
    b, n, h, w = ox.shape
    if ptype == "horizontal":
        z = jnp.zeros((b, n, h, 1), jnp.float32)
        ox2 = jnp.concatenate([
            jnp.concatenate([z, ox[:, :, :, :-1]], axis=3), ox,
            jnp.concatenate([ox[:, :, :, 1:], z], axis=3)], axis=1)
        oy2 = jnp.concatenate([
            jnp.concatenate([z, oy[:, :, :, :-1]], axis=3), oy,
            jnp.concatenate([oy[:, :, :, 1:], z], axis=3)], axis=1)
    else:
        z = jnp.zeros((b, n, 1, w), jnp.float32)
        ox2 = jnp.concatenate([
            jnp.concatenate([z, ox[:, :, :-1, :]], axis=2), ox,
            jnp.concatenate([ox[:, :, 1:, :], z], axis=2)], axis=1)
        oy2 = jnp.concatenate([
            jnp.concatenate([z, oy[:, :, :-1, :]], axis=2), oy,
            jnp.concatenate([oy[:, :, 1:, :], z], axis=2)], axis=1)
    return ox2, oy2


def _offsets_to_rows(ox, oy):
    base_x = jnp.arange(W).reshape(1, 1, 1, W)
    base_y = jnp.arange(H).reshape(1, 1, H, 1)
    h_ = jnp.clip(base_y + oy, 0, H - 1)
    w_ = jnp.clip(base_x + ox, 0, W - 1)
    inds = h_ * W + w_
    m = inds.reshape(B, NCAND, HW).transpose(0, 2, 1).astype(jnp.int32)
    g = m + (jnp.arange(B, dtype=jnp.int32) * HW).reshape(B, 1, 1)
    return g.reshape(P, NCAND)


def _rows_to_offsets(m8):
    m = m8.reshape(B, HW, K).transpose(0, 2, 1).reshape(B, K, H, W)
    base_x = jnp.arange(W).reshape(1, 1, 1, W)
    base_y = jnp.arange(H).reshape(1, 1, H, 1)
    ox = m % W - base_x
    oy = m // W - base_y
    return ox, oy


def kernel(left_features, right_features, offset_x, offset_y):
    rt = right_features.T.reshape(P, C)
    leftr = left_features.transpose(0, 2, 1).reshape(P, C)

    ox, oy = _propagation(offset_x, offset_y, "vertical")
    rows = _offsets_to_rows(ox, oy)
    _, inds16 = _sc_evaluate(rt, rows.reshape(-1, 2, HROWS), leftr)
    ox, oy = _rows_to_offsets(inds16[:, :K])

    ox, oy = _propagation(ox, oy, "horizontal")
    rows = _offsets_to_rows(ox, oy)
    corr16, inds16 = _sc_evaluate(rt, rows.reshape(-1, 2, HROWS), leftr)
    ox, oy = _rows_to_offsets(inds16[:, :K])
    corr = corr16[:, :K].reshape(B, HW, K).transpose(0, 2, 1)
    return ox, oy, corr


# speedup vs baseline: 1.9818x; 1.9818x over previous
import jax
import jax.numpy as jnp
from jax import lax
from jax.experimental import pallas as pl
from jax.experimental.pallas import tpu as pltpu
from jax.experimental.pallas import tpu_sc as plsc

B, C, H, W, NUM = 4, 256, 64, 64, 8
HW = H * W
P = B * HW
NCAND = 3 * NUM
K = NCAND // 3
NWORKERS = 32
PIX_PER_W = P // NWORKERS
CHUNK = 8
NCHUNKS = PIX_PER_W // CHUNK
ROWS = CHUNK * NCAND
HROWS = ROWS // 2
NEG = -1e30
FLT_MIN = 1.1754944e-38
LOG_FLT_MIN = -87.336545


def _sc_evaluate_kernel(rt_hbm, inds_hbm, left_hbm, corr_out, inds_out,
                        idx3, rows2, left2,
                        outc_v, outi_v, sem0, sem1):
    wid = lax.axis_index("s") * 2 + lax.axis_index("c")
    bbase = (wid // 8) * HW
    iota16 = lax.iota(jnp.int32, 16)
    sems = (sem0, sem1)

    def issue(ch, b):
        pix_base = wid * PIX_PER_W + ch * CHUNK
        pltpu.sync_copy(inds_hbm.at[wid * NCHUNKS + ch], idx3.at[b])
        pltpu.async_copy(left_hbm.at[pl.ds(pix_base, CHUNK)], left2.at[b],
                         sems[b])
        pltpu.async_copy(rt_hbm.at[idx3.at[b].at[0]],
                         rows2.at[b].at[pl.ds(0, HROWS)], sems[b])
        pltpu.async_copy(rt_hbm.at[idx3.at[b].at[1]],
                         rows2.at[b].at[pl.ds(HROWS, HROWS)], sems[b])

    def drain(b):
        pltpu.make_async_copy(left_hbm.at[pl.ds(0, CHUNK)], left2.at[b],
                              sems[b]).wait()
        pltpu.make_async_copy(rt_hbm.at[idx3.at[b].at[0]],
                              rows2.at[b].at[pl.ds(0, HROWS)], sems[b]).wait()
        pltpu.make_async_copy(rt_hbm.at[idx3.at[b].at[1]],
                              rows2.at[b].at[pl.ds(HROWS, HROWS)],
                              sems[b]).wait()

    def compute(ch, b):
        pix_base = wid * PIX_PER_W + ch * CHUNK

        def pix_body(p, carry2):
            lvecs = [left2[b, p, pl.ds(v * 16, 16)] for v in range(16)]
            c0 = jnp.zeros((16,), jnp.float32)
            c1 = jnp.full((16,), NEG, jnp.float32)
            for cand in range(NCAND):
                row = p * NCAND + cand
                acc = lvecs[0] * rows2[b, row, pl.ds(0, 16)]
                for v in range(1, 16):
                    acc = acc + lvecs[v] * rows2[b, row, pl.ds(v * 16, 16)]
                s = jnp.sum(acc) * 100.0
                if cand < 16:
                    c0 = jnp.where(iota16 == cand, s, c0)
                else:
                    c1 = jnp.where(iota16 == cand - 16, s, c1)
            m = jnp.max(jnp.maximum(c0, c1))
            d0 = c0 - m
            d1 = c1 - m
            e0 = jnp.where(d0 < LOG_FLT_MIN, 0.0, jnp.exp(d0))
            e1 = jnp.where(d1 < LOG_FLT_MIN, 0.0, jnp.exp(d1))
            svec = jnp.zeros((16,), jnp.float32) + jnp.sum(e0 + e1)
            p0 = e0 / svec
            p1 = e1 / svec
            p0 = jnp.where(p0 < FLT_MIN, 0.0, p0)
            p1 = jnp.where(p1 < FLT_MIN, 0.0, p1)

            topv = jnp.zeros((16,), jnp.float32)
            topi = jnp.zeros((16,), jnp.int32)
            for k in range(K):
                mx = jnp.max(jnp.maximum(p0, p1))
                g0 = jnp.where(p0 == mx, iota16, 64)
                g1 = jnp.where(p1 == mx, iota16 + 16, 64)
                idx = jnp.min(jnp.minimum(g0, g1))
                topv = jnp.where(iota16 == k, mx, topv)
                topi = jnp.where(iota16 == k, idx, topi)
                p0 = jnp.where(iota16 == idx, -1.0, p0)
                p1 = jnp.where(iota16 == idx - 16, -1.0, p1)

            f = p * NCAND + topi
            ga = plsc.load_gather(idx3.at[b].at[0],
                                  [jnp.minimum(f, HROWS - 1)])
            gb = plsc.load_gather(idx3.at[b].at[1],
                                  [jnp.maximum(f - HROWS, 0)])
            g = jnp.where(f < HROWS, ga, gb)
            outi_v[p, :] = (g - bbase).astype(jnp.float32)
            outc_v[p, :] = topv
            return carry2

        lax.fori_loop(0, CHUNK, pix_body, 0)
        pltpu.sync_copy(outc_v, corr_out.at[pl.ds(pix_base, CHUNK)])
        pltpu.sync_copy(outi_v, inds_out.at[pl.ds(pix_base, CHUNK)])

    issue(0, 0)

    def pair_body(gidx, carry):
        che = 2 * gidx
        issue(che + 1, 1)
        drain(0)
        compute(che, 0)
        issue(jnp.minimum(che + 2, NCHUNKS - 2), 0)
        drain(1)
        compute(che + 1, 1)
        return carry

    lax.fori_loop(0, NCHUNKS // 2, pair_body, 0)
    drain(0)


@jax.jit
def _sc_evaluate(rt, inds_flat, leftr):
    mesh = plsc.VectorSubcoreMesh(core_axis_name="c", subcore_axis_name="s")
    fn = pl.kernel(
        _sc_evaluate_kernel,
        mesh=mesh,
        compiler_params=pltpu.CompilerParams(needs_layout_passes=False),
        out_type=[
            jax.ShapeDtypeStruct((P, 16), jnp.float32),
            jax.ShapeDtypeStruct((P, 16), jnp.float32),
        ],
        scratch_types=[
            pltpu.VMEM((2, 2, HROWS), jnp.int32),
            pltpu.VMEM((2, ROWS, C), jnp.float32),
            pltpu.VMEM((2, CHUNK, C), jnp.float32),
            pltpu.VMEM((CHUNK, 16), jnp.float32),
            pltpu.VMEM((CHUNK, 16), jnp.float32),
            pltpu.SemaphoreType.DMA,
            pltpu.SemaphoreType.DMA,
        ],
    )
    return fn(rt, inds_flat, leftr)


def _propagation(ox, oy, ptype):
    b, n, h, w = ox.shape
    if ptype == "horizontal":
        z = jnp.zeros((b, n, h, 1), jnp.float32)
        ox2 = jnp.concatenate([
            jnp.concatenate([z, ox[:, :, :, :-1]], axis=3), ox,
            jnp.concatenate([ox[:, :, :, 1:], z], axis=3)], axis=1)
        oy2 = jnp.concatenate([
            jnp.concatenate([z, oy[:, :, :, :-1]], axis=3), oy,
            jnp.concatenate([oy[:, :, :, 1:], z], axis=3)], axis=1)
    else:
        z = jnp.zeros((b, n, 1, w), jnp.float32)
        ox2 = jnp.concatenate([
            jnp.concatenate([z, ox[:, :, :-1, :]], axis=2), ox,
            jnp.concatenate([ox[:, :, 1:, :], z], axis=2)], axis=1)
        oy2 = jnp.concatenate([
            jnp.concatenate([z, oy[:, :, :-1, :]], axis=2), oy,
            jnp.concatenate([oy[:, :, 1:, :], z], axis=2)], axis=1)
    return ox2, oy2


def _offsets_to_rows(ox, oy):
    base_x = jnp.arange(W).reshape(1, 1, 1, W)
    base_y = jnp.arange(H).reshape(1, 1, H, 1)
    h_ = jnp.clip(base_y + oy, 0, H - 1)
    w_ = jnp.clip(base_x + ox, 0, W - 1)
    inds = h_ * W + w_
    m = inds.reshape(B, NCAND, HW).transpose(0, 2, 1).astype(jnp.int32)
    g = m + (jnp.arange(B, dtype=jnp.int32) * HW).reshape(B, 1, 1)
    return g.reshape(P, NCAND)


def _rows_to_offsets(m8):
    m = m8.reshape(B, HW, K).transpose(0, 2, 1).reshape(B, K, H, W)
    base_x = jnp.arange(W).reshape(1, 1, 1, W)
    base_y = jnp.arange(H).reshape(1, 1, H, 1)
    ox = m % W - base_x
    oy = m // W - base_y
    return ox, oy


def kernel(left_features, right_features, offset_x, offset_y):
    rt = right_features.T.reshape(P, C)
    leftr = left_features.transpose(0, 2, 1).reshape(P, C)

    ox, oy = _propagation(offset_x, offset_y, "vertical")
    rows = _offsets_to_rows(ox, oy)
    _, inds16 = _sc_evaluate(rt, rows.reshape(-1, 2, HROWS), leftr)
    ox, oy = _rows_to_offsets(inds16[:, :K])

    ox, oy = _propagation(ox, oy, "horizontal")
    rows = _offsets_to_rows(ox, oy)
    corr16, inds16 = _sc_evaluate(rt, rows.reshape(-1, 2, HROWS), leftr)
    ox, oy = _rows_to_offsets(inds16[:, :K])
    corr = corr16[:, :K].reshape(B, HW, K).transpose(0, 2, 1)
    return ox, oy, corr
